# baseline (speedup 1.0000x reference)
import functools

import numpy as np
import jax
import jax.numpy as jnp
from jax import lax
from jax.experimental import pallas as pl
from jax.experimental.pallas import tpu as pltpu
from jax.experimental.pallas import tpu_sc as plsc

_MAX_LEN = 2048
_D_MODEL = 1024

_NC, _NS, _L = 2, 16, 16
_NW = _NC * _NS
_PPC = 8
_NB = 3


def _pe_table(max_len, d_model):
    pos = np.arange(0, max_len, dtype=np.float64)[:, None]
    mul = np.exp(np.arange(0, d_model, 2, dtype=np.float64)
                 * -(np.log(10000.0) / d_model))
    pe = np.zeros((max_len, d_model), dtype=np.float64)
    pe[:, 0::2] = np.sin(pos * mul)
    pe[:, 1::2] = np.cos(pos * mul)
    return jnp.asarray(pe, dtype=jnp.float32)


_PE = _pe_table(_MAX_LEN, _D_MODEL).reshape(-1)


def kernel(tokens, embed_table):
    B, S = tokens.shape
    V, D = embed_table.shape
    n_tok = B * S
    ppw = S // _NW
    n_chunks = ppw // _PPC
    rows_c = B * _PPC
    groups = D // _L
    half = groups // 2

    gidx = (tokens.astype(jnp.int32)
            .reshape(B, _NW, n_chunks, _PPC)
            .transpose(1, 2, 0, 3)
            .reshape(_NW, n_chunks, rows_c))
    b_ix = np.arange(B)[None, None, :, None]
    w_ix = np.arange(_NW)[:, None, None, None]
    c_ix = np.arange(n_chunks)[None, :, None, None]
    i_ix = np.arange(_PPC)[None, None, None, :]
    sidx = jnp.asarray(
        (b_ix * S + w_ix * ppw + c_ix * _PPC + i_ix)
        .reshape(_NW, n_chunks, rows_c).astype(np.int32))

    mesh = plsc.VectorSubcoreMesh(core_axis_name="c", subcore_axis_name="s")

    @functools.partial(
        pl.kernel,
        mesh=mesh,
        out_type=jax.ShapeDtypeStruct((n_tok, D), jnp.float32),
        scratch_types=(
            [pltpu.VMEM((n_chunks, rows_c), jnp.int32)] * 2
            + [pltpu.VMEM((rows_c, D), jnp.float32)] * _NB
            + [pltpu.VMEM((_PPC * D,), jnp.float32)] * _NB
            + [pltpu.SemaphoreType.DMA] * (3 * _NB)
        ),
    )
    def emb_kernel(table_h, gidx_h, sidx_h, pe_h, out_h,
                   gidx_v, sidx_v, *scr):
        rows = list(scr[:_NB])
        pes = list(scr[_NB:2 * _NB])
        gsem = list(scr[2 * _NB:3 * _NB])
        psem = list(scr[3 * _NB:4 * _NB])
        ssem = list(scr[4 * _NB:5 * _NB])
        wid = lax.axis_index("s") * _NC + lax.axis_index("c")
        pltpu.sync_copy(gidx_h.at[wid], gidx_v)
        pltpu.sync_copy(sidx_h.at[wid], sidx_v)
        pbase = wid * ppw

        def start_chunk(c):
            b = c % _NB
            pltpu.async_copy(table_h.at[gidx_v.at[c]], rows[b], gsem[b])
            pltpu.async_copy(pe_h.at[pl.ds((pbase + c * _PPC) * D, _PPC * D)],
                             pes[b], psem[b])

        def wait_chunk(c):
            b = c % _NB
            pltpu.make_async_copy(table_h.at[gidx_v.at[c]], rows[b],
                                  gsem[b]).wait()
            pltpu.make_async_copy(
                pe_h.at[pl.ds((pbase + c * _PPC) * D, _PPC * D)],
                pes[b], psem[b]).wait()

        def start_scatter(c):
            b = c % _NB
            pltpu.async_copy(rows[b], out_h.at[sidx_v.at[c]], ssem[b])

        def wait_scatter(c):
            b = c % _NB
            pltpu.make_async_copy(rows[b], out_h.at[sidx_v.at[c]],
                                  ssem[b]).wait()

        def add_chunk(c):
            b = c % _NB
            rv, pv = rows[b], pes[b]

            def body(t, _):
                i = t >> 1
                base = (t & 1) * (half * _L)
                for jg in range(half):
                    off = base + jg * _L
                    pe_reg = pv[pl.ds(i * D + off, _L)]
                    for bb in range(B):
                        r = bb * _PPC + i
                        rv[r, pl.ds(off, _L)] = rv[r, pl.ds(off, _L)] + pe_reg
                return 0

            lax.fori_loop(0, _PPC * 2, body, 0)

        start_chunk(0)
        start_chunk(1)
        for c in range(n_chunks):
            wait_chunk(c)
            add_chunk(c)
            start_scatter(c)
            if c + 2 < n_chunks:
                if c >= 1:
                    wait_scatter(c - 1)
                start_chunk(c + 2)
        for c in range(n_chunks - _NB, n_chunks):
            wait_scatter(c)

    out = emb_kernel(embed_table, gidx, sidx, _PE)
    return out.reshape(B, S, D)


# speedup vs baseline: 1.0477x; 1.0477x over previous
import functools

import numpy as np
import jax
import jax.numpy as jnp
from jax import lax
from jax.experimental import pallas as pl
from jax.experimental.pallas import tpu as pltpu
from jax.experimental.pallas import tpu_sc as plsc

_MAX_LEN = 2048
_D_MODEL = 1024

_NC, _NS, _L = 2, 16, 16
_NW = _NC * _NS
_PPC = 8
_NB = 3


def _pe_table(max_len, d_model):
    pos = np.arange(0, max_len, dtype=np.float64)[:, None]
    mul = np.exp(np.arange(0, d_model, 2, dtype=np.float64)
                 * -(np.log(10000.0) / d_model))
    pe = np.zeros((max_len, d_model), dtype=np.float64)
    pe[:, 0::2] = np.sin(pos * mul)
    pe[:, 1::2] = np.cos(pos * mul)
    return jnp.asarray(pe, dtype=jnp.float32)


_PE = _pe_table(_MAX_LEN, _D_MODEL)


def kernel(tokens, embed_table):
    B, S = tokens.shape
    V, D = embed_table.shape
    n_tok = B * S
    ppw = S // _NW
    n_chunks = ppw // _PPC
    rows_c = B * _PPC
    groups = D // _L
    half = groups // 2

    gidx = (tokens.astype(jnp.int32)
            .reshape(B, _NW, n_chunks, _PPC)
            .transpose(1, 2, 0, 3)
            .reshape(_NW, n_chunks, rows_c))
    b_ix = np.arange(B)[None, None, :, None]
    w_ix = np.arange(_NW)[:, None, None, None]
    c_ix = np.arange(n_chunks)[None, :, None, None]
    i_ix = np.arange(_PPC)[None, None, None, :]
    sidx = jnp.asarray(
        (b_ix * S + w_ix * ppw + c_ix * _PPC + i_ix)
        .reshape(_NW, n_chunks, rows_c).astype(np.int32))

    mesh = plsc.VectorSubcoreMesh(core_axis_name="c", subcore_axis_name="s")

    @functools.partial(
        pl.kernel,
        mesh=mesh,
        out_type=jax.ShapeDtypeStruct((n_tok, D), jnp.float32),
        scratch_types=(
            [pltpu.VMEM((n_chunks, rows_c), jnp.int32)] * 2
            + [pltpu.VMEM((rows_c, D), jnp.float32)] * _NB
            + [pltpu.VMEM((_PPC, D), jnp.float32)] * _NB
            + [pltpu.SemaphoreType.DMA] * (3 * _NB)
        ),
    )
    def emb_kernel(table_h, gidx_h, sidx_h, pe_h, out_h,
                   gidx_v, sidx_v, *scr):
        rows = list(scr[:_NB])
        pes = list(scr[_NB:2 * _NB])
        gsem = list(scr[2 * _NB:3 * _NB])
        psem = list(scr[3 * _NB:4 * _NB])
        ssem = list(scr[4 * _NB:5 * _NB])
        wid = lax.axis_index("s") * _NC + lax.axis_index("c")
        pltpu.sync_copy(gidx_h.at[wid], gidx_v)
        pltpu.sync_copy(sidx_h.at[wid], sidx_v)
        pbase = wid * ppw

        def start_chunk(c):
            b = c % _NB
            pltpu.async_copy(table_h.at[gidx_v.at[c]], rows[b], gsem[b])
            pltpu.async_copy(pe_h.at[pl.ds(pbase + c * _PPC, _PPC)],
                             pes[b], psem[b])

        def wait_chunk(c):
            b = c % _NB
            pltpu.make_async_copy(table_h.at[gidx_v.at[c]], rows[b],
                                  gsem[b]).wait()
            pltpu.make_async_copy(pe_h.at[pl.ds(pbase + c * _PPC, _PPC)],
                                  pes[b], psem[b]).wait()

        def start_scatter(c):
            b = c % _NB
            pltpu.async_copy(rows[b], out_h.at[sidx_v.at[c]], ssem[b])

        def wait_scatter(c):
            b = c % _NB
            pltpu.make_async_copy(rows[b], out_h.at[sidx_v.at[c]],
                                  ssem[b]).wait()

        def add_chunk(c):
            b = c % _NB
            rv, pv = rows[b], pes[b]

            def body(t, _):
                i = t >> 1
                base = (t & 1) * (half * _L)
                for jg in range(half):
                    off = base + jg * _L
                    pe_reg = pv[i, pl.ds(off, _L)]
                    for bb in range(B):
                        r = bb * _PPC + i
                        rv[r, pl.ds(off, _L)] = rv[r, pl.ds(off, _L)] + pe_reg
                return 0

            lax.fori_loop(0, _PPC * 2, body, 0)

        start_chunk(0)
        start_chunk(1)
        for c in range(n_chunks):
            wait_chunk(c)
            add_chunk(c)
            start_scatter(c)
            if c + 2 < n_chunks:
                if c >= 1:
                    wait_scatter(c - 1)
                start_chunk(c + 2)
        for c in range(n_chunks - _NB, n_chunks):
            wait_scatter(c)

    out = emb_kernel(embed_table, gidx, sidx, _PE)
    return out.reshape(B, S, D)
